# baseline (speedup 1.0000x reference)
import functools

import jax
import jax.numpy as jnp
from jax import lax
from jax.experimental import pallas as pl
from jax.experimental.pallas import tpu as pltpu
from jax.experimental.pallas import tpu_sc as plsc

N = 10000
E = 320000
D = 128
H = 4
C = 32

NPAD = 10240
ROWS = 512

EW = E // 16
BLK = 400
NBLK = EW // BLK
NGRP = BLK // 16
NP1 = 10000
F32 = jnp.float32
I32 = jnp.int32


def _proj_body(x_ref, w_ref, y_ref):
    y_ref[...] = jnp.dot(x_ref[...], w_ref[...],
                         preferred_element_type=F32)


def _project(xpad, wcat):
    return pl.pallas_call(
        _proj_body,
        grid=(NPAD // ROWS,),
        in_specs=[
            pl.BlockSpec((ROWS, D), lambda i: (i, 0)),
            pl.BlockSpec((D, 256), lambda i: (0, 0)),
        ],
        out_specs=pl.BlockSpec((ROWS, 256), lambda i: (i, 0)),
        out_shape=jax.ShapeDtypeStruct((NPAD, 256), F32),
    )(xpad, wcat)


def _sc_body(esrc, edst, sdt, whr, mtab, iden, outc,
             agg, den, tabs, dloc, rows, rows2,
             sbuf, dbuf, gibuf, sibuf, abuf, mbuf, ribuf):
    cid = lax.axis_index("c")
    sid = lax.axis_index("s")
    iota16 = lax.iota(I32, 16)
    zf = jnp.zeros((16,), F32)

    def zrows(r, c):
        rows[r, pl.ds(0, 16)] = zf
        rows[r, pl.ds(16, 16)] = zf
        return c
    lax.fori_loop(0, BLK, zrows, 0)

    def zd(r, c):
        dloc[r, pl.ds(0, 16)] = zf
        return c
    lax.fori_loop(0, 1280, zd, 0)

    pltpu.sync_copy(iden.at[pl.ds(0, 640)], ribuf)
    for ch in range(5):
        pltpu.sync_copy(rows.at[pl.ds(0, 256)],
                        agg.at[pl.ds(sid * 1280 + ch * 256, 256)])
    pltpu.sync_copy(dloc.at[pl.ds(0, 80)], den.at[pl.ds(sid * 80, 80)])
    plsc.subcore_barrier()

    g0 = 2 * cid
    pltpu.sync_copy(sdt.at[pl.ds(g0 * N, N)], tabs.at[pl.ds(0, N)])
    pltpu.sync_copy(sdt.at[pl.ds((g0 + 4) * N, N)], tabs.at[pl.ds(NP1, N)])
    pltpu.sync_copy(sdt.at[pl.ds((g0 + 1) * N, N)], tabs.at[pl.ds(2 * NP1, N)])
    pltpu.sync_copy(sdt.at[pl.ds((g0 + 5) * N, N)], tabs.at[pl.ds(3 * NP1, N)])
    pltpu.sync_copy(mtab.at[pl.ds(g0 * 16, 32)], mbuf)
    m0 = mbuf[pl.ds(0, 16)]
    m1 = mbuf[pl.ds(16, 16)]
    ebase = sid * EW

    def p1_block(b, c):
        base = ebase + b * BLK
        pltpu.sync_copy(esrc.at[pl.ds(base, BLK)], sbuf)
        pltpu.sync_copy(edst.at[pl.ds(base, BLK)], dbuf)

        def p1_group(g, cc):
            off = g * 16
            sv = sbuf[pl.ds(off, 16)]
            dv = dbuf[pl.ds(off, 16)]
            for lh in range(2):
                se = plsc.load_gather(tabs, [sv + 2 * lh * NP1])
                de = plsc.load_gather(tabs, [dv + (2 * lh + 1) * NP1])
                z = se + de
                e = jnp.maximum(z, 0.2 * z)
                ee = jnp.exp(e - (m0 if lh == 0 else m1))
                didx = dv + lh * NP1
                rowi = lax.shift_right_logical(didx, 4)
                coli = jnp.bitwise_and(didx, 15)
                plsc.addupdate_scatter(dloc, [rowi, coli], ee)
            return cc
        return c
    lax.fori_loop(0, NBLK, p1_block, 0)

    pltpu.sync_copy(dloc.at[pl.ds(0, 640)], den.at[ribuf], add=True)
    pltpu.sync_copy(iden.at[pl.ds(640, 640)], ribuf)
    pltpu.sync_copy(dloc.at[pl.ds(640, 640)], den.at[ribuf], add=True)
    plsc.subcore_barrier()
    pltpu.sync_copy(den, dloc)

    def p2_block(b, c):
        base = ebase + b * BLK
        pltpu.sync_copy(esrc.at[pl.ds(base, BLK)], sbuf)
        pltpu.sync_copy(edst.at[pl.ds(base, BLK)], dbuf)
        for lh in range(2):
            def p2_idx(g, cc):
                off = g * 16
                sv = sbuf[pl.ds(off, 16)]
                dv = dbuf[pl.ds(off, 16)]
                se = plsc.load_gather(tabs, [sv + 2 * lh * NP1])
                de = plsc.load_gather(tabs, [dv + (2 * lh + 1) * NP1])
                z = se + de
                e = jnp.maximum(z, 0.2 * z)
                ee = jnp.exp(e - (m0 if lh == 0 else m1))
                didx = dv + lh * NP1
                dnv = plsc.load_gather(
                    dloc, [lax.shift_right_logical(didx, 4),
                           jnp.bitwise_and(didx, 15)])
                al = ee / (dnv + 1e-16)
                abuf[pl.ds(off, 16)] = al
                gibuf[pl.ds(off, 16)] = sv * H + (g0 + lh)
                sibuf[pl.ds(off, 16)] = dv * 2 + lh
                return cc
            pltpu.sync_copy(whr.at[gibuf], rows)

            def p2_scale(g, cc):
                al = abuf[pl.ds(g * 16, 16)]
                base = g * 16
                for k in range(16):
                    spl = lax.gather(
                        al, jnp.full((16, 1), k, I32),
                        lax.GatherDimensionNumbers(
                            offset_dims=(), collapsed_slice_dims=(0,),
                            start_index_map=(0,)),
                        (1,), mode=lax.GatherScatterMode.PROMISE_IN_BOUNDS)
                    v0 = rows[base + k, pl.ds(0, 16)]
                    v1 = rows[base + k, pl.ds(16, 16)]
                    rows2[base + k, pl.ds(0, 16)] = v0 * spl
                    rows2[base + k, pl.ds(16, 16)] = v1 * spl
                return cc
            lax.fori_loop(0, NGRP, p2_scale, 0)
            pltpu.sync_copy(rows2, agg.at[sibuf], add=True)
        return c
    lax.fori_loop(0, NBLK, p2_block, 0)
    plsc.subcore_barrier()

    nbase = sid * 1280
    for ch in range(5):
        rb = nbase + ch * 256
        pltpu.sync_copy(agg.at[pl.ds(rb, 256)], rows.at[pl.ds(0, 256)])

        def elu_loop(r, c):
            for half in range(2):
                v = rows[r, pl.ds(half * 16, 16)]
                ev = jnp.exp(v) - 1.0
                rows[r, pl.ds(half * 16, 16)] = jnp.where(v > 0.0, v, ev)
            return c
        lax.fori_loop(0, 256, elu_loop, 0)
        pltpu.sync_copy(rows.at[pl.ds(0, 256)], outc.at[cid, pl.ds(rb, 256)])


_sc_kernel = functools.partial(
    pl.kernel,
    out_type=jax.ShapeDtypeStruct((2, 20480, C), F32),
    mesh=plsc.VectorSubcoreMesh(core_axis_name="c", subcore_axis_name="s"),
    compiler_params=pltpu.CompilerParams(needs_layout_passes=False, use_tc_tiling_on_sc=False),
    scratch_types=[
        pltpu.VMEM_SHARED((20480, C), F32),
        pltpu.VMEM_SHARED((1280, 16), F32),
        pltpu.VMEM((4 * NP1,), F32),
        pltpu.VMEM((1280, 16), F32),
        pltpu.VMEM((BLK, C), F32),
        pltpu.VMEM((BLK, C), F32),
        pltpu.VMEM((BLK,), I32),
        pltpu.VMEM((BLK,), I32),
        pltpu.VMEM((BLK,), I32),
        pltpu.VMEM((BLK,), I32),
        pltpu.VMEM((BLK,), F32),
        pltpu.VMEM((32,), F32),
        pltpu.VMEM((640,), I32),
    ],
)(_sc_body)


def kernel(x, edge_index, W, att):
    att_s = att[0, :, :C]
    att_d = att[0, :, C:]
    W3 = W.reshape(D, H, C)
    Ws = (W3 * att_s[None, :, :]).sum(-1)
    Wd = (W3 * att_d[None, :, :]).sum(-1)
    wcat = jnp.concatenate(
        [W, Ws, Wd, jnp.zeros((D, 256 - D - 2 * H), F32)], axis=1)
    xpad = jnp.concatenate([x, jnp.zeros((NPAD - N, D), F32)], axis=0)

    y = _project(xpad, wcat)
    wh = y[:N, :D]
    s = y[:N, D:D + H]
    d = y[:N, D + H:D + 2 * H]

    sdt = jnp.concatenate([s.T, d.T], axis=0).reshape(8 * N)
    mh = s.max(0) + d.max(0)
    mh = jnp.maximum(mh, 0.2 * mh)
    mtab = jnp.repeat(mh, 16)
    whr = wh.reshape(N * H, C)
    iden = jnp.arange(1280, dtype=I32)

    outc = _sc_kernel(edge_index[0], edge_index[1], sdt, whr, mtab, iden)
    return jnp.concatenate(
        [outc[0, :2 * N].reshape(N, 2 * C),
         outc[1, :2 * N].reshape(N, 2 * C)], axis=1)


# speedup vs baseline: 23.7404x; 23.7404x over previous
import functools

import jax
import jax.numpy as jnp
from jax import lax
from jax.experimental import pallas as pl
from jax.experimental.pallas import tpu as pltpu
from jax.experimental.pallas import tpu_sc as plsc

N = 10000
E = 320000
D = 128
H = 4
C = 32

NPAD = 10240
ROWS = 512

EW = E // 16
BLK = 400
NGRP = BLK // 16
SBLK = 2000
NSB = EW // SBLK
NSUB = SBLK // BLK
NP1 = 10000
F32 = jnp.float32
I32 = jnp.int32


def _proj_body(x_ref, w_ref, y_ref):
    y_ref[...] = jnp.dot(x_ref[...], w_ref[...],
                         preferred_element_type=F32)


def _project(xpad, wcat):
    return pl.pallas_call(
        _proj_body,
        grid=(NPAD // ROWS,),
        in_specs=[
            pl.BlockSpec((ROWS, D), lambda i: (i, 0)),
            pl.BlockSpec((D, 256), lambda i: (0, 0)),
        ],
        out_specs=pl.BlockSpec((ROWS, 256), lambda i: (i, 0)),
        out_shape=jax.ShapeDtypeStruct((NPAD, 256), F32),
    )(xpad, wcat)


def _sc_body(esrc, edst, sdt, whr, mtab, iden, outc,
             agg, den, tabs, dloc, rows,
             sbuf, dbuf, gibuf, sibuf, abuf, mbuf, ribuf):
    cid = lax.axis_index("c")
    sid = lax.axis_index("s")
    iota16 = lax.iota(I32, 16)
    zf = jnp.zeros((16,), F32)

    def zrows(r, c):
        rows[r, pl.ds(0, 16)] = zf
        rows[r, pl.ds(16, 16)] = zf
        return c
    lax.fori_loop(0, BLK, zrows, 0)

    def zd(r, c):
        dloc[r, pl.ds(0, 16)] = zf
        return c
    lax.fori_loop(0, 1280, zd, 0)

    pltpu.sync_copy(iden.at[pl.ds(0, 640)], ribuf)
    for ch in range(5):
        pltpu.sync_copy(rows.at[pl.ds(0, 256)],
                        agg.at[pl.ds(sid * 1280 + ch * 256, 256)])
    pltpu.sync_copy(dloc.at[pl.ds(0, 80)], den.at[pl.ds(sid * 80, 80)])
    plsc.subcore_barrier()

    g0 = 2 * cid
    pltpu.sync_copy(sdt.at[pl.ds(g0 * N, N)], tabs.at[pl.ds(0, N)])
    pltpu.sync_copy(sdt.at[pl.ds((g0 + 4) * N, N)], tabs.at[pl.ds(NP1, N)])
    pltpu.sync_copy(sdt.at[pl.ds((g0 + 1) * N, N)], tabs.at[pl.ds(2 * NP1, N)])
    pltpu.sync_copy(sdt.at[pl.ds((g0 + 5) * N, N)], tabs.at[pl.ds(3 * NP1, N)])
    pltpu.sync_copy(mtab.at[pl.ds(g0 * 16, 32)], mbuf)
    m0 = mbuf[pl.ds(0, 16)]
    m1 = mbuf[pl.ds(16, 16)]
    ebase = sid * EW

    def p1_block(b, c):
        base = ebase + b * SBLK
        pltpu.sync_copy(esrc.at[pl.ds(base, SBLK)], sbuf)
        pltpu.sync_copy(edst.at[pl.ds(base, SBLK)], dbuf)

        def p1_group(g, cc):
            off = g * 16
            sv = sbuf[pl.ds(off, 16)]
            dv = dbuf[pl.ds(off, 16)]
            for lh in range(2):
                se = plsc.load_gather(tabs, [sv + 2 * lh * NP1])
                de = plsc.load_gather(tabs, [dv + (2 * lh + 1) * NP1])
                z = se + de
                e = jnp.maximum(z, 0.2 * z)
                ee = jnp.exp(e - (m0 if lh == 0 else m1))
                didx = dv + lh * NP1
                rowi = lax.shift_right_logical(didx, 4)
                coli = jnp.bitwise_and(didx, 15)
                plsc.addupdate_scatter(dloc, [rowi, coli], ee)
            return cc
        lax.fori_loop(0, SBLK // 16, p1_group, 0)
        return c
    lax.fori_loop(0, NSB, p1_block, 0)

    pltpu.sync_copy(dloc.at[pl.ds(0, 640)], den.at[ribuf], add=True)
    pltpu.sync_copy(iden.at[pl.ds(640, 640)], ribuf)
    pltpu.sync_copy(dloc.at[pl.ds(640, 640)], den.at[ribuf], add=True)
    plsc.subcore_barrier()
    pltpu.sync_copy(den, dloc)

    def p2_block(b, c):
        base = ebase + b * SBLK
        pltpu.sync_copy(esrc.at[pl.ds(base, SBLK)], sbuf)
        pltpu.sync_copy(edst.at[pl.ds(base, SBLK)], dbuf)

        def p2_sub(u, c2):
          for lh in range(2):
            def p2_idx(g, cc):
                off = u * BLK + g * 16
                sv = sbuf[pl.ds(off, 16)]
                dv = dbuf[pl.ds(off, 16)]
                se = plsc.load_gather(tabs, [sv + 2 * lh * NP1])
                de = plsc.load_gather(tabs, [dv + (2 * lh + 1) * NP1])
                z = se + de
                e = jnp.maximum(z, 0.2 * z)
                ee = jnp.exp(e - (m0 if lh == 0 else m1))
                didx = dv + lh * NP1
                dnv = plsc.load_gather(
                    dloc, [lax.shift_right_logical(didx, 4),
                           jnp.bitwise_and(didx, 15)])
                al = ee / (dnv + 1e-16)
                abuf[pl.ds(g * 16, 16)] = al
                gibuf[pl.ds(g * 16, 16)] = sv * H + (g0 + lh)
                sibuf[pl.ds(g * 16, 16)] = dv * 2 + lh
                return cc
            lax.fori_loop(0, NGRP, p2_idx, 0)
            pltpu.sync_copy(whr.at[gibuf], rows)

            def p2_scale(g, cc):
                al = abuf[pl.ds(g * 16, 16)]
                base = g * 16
                for k in range(16):
                    spl = lax.gather(
                        al, jnp.full((16, 1), k, I32),
                        lax.GatherDimensionNumbers(
                            offset_dims=(), collapsed_slice_dims=(0,),
                            start_index_map=(0,)),
                        (1,), mode=lax.GatherScatterMode.PROMISE_IN_BOUNDS)
                    v0 = rows[base + k, pl.ds(0, 16)]
                    v1 = rows[base + k, pl.ds(16, 16)]
                    rows[base + k, pl.ds(0, 16)] = v0 * spl
                    rows[base + k, pl.ds(16, 16)] = v1 * spl
                return cc
            lax.fori_loop(0, NGRP, p2_scale, 0)
            pltpu.sync_copy(rows, agg.at[sibuf], add=True)
          return c2
        lax.fori_loop(0, NSUB, p2_sub, 0)
        return c
    lax.fori_loop(0, NSB, p2_block, 0)
    plsc.subcore_barrier()

    nbase = sid * 1280
    for ch in range(5):
        rb = nbase + ch * 256
        pltpu.sync_copy(agg.at[pl.ds(rb, 256)], rows.at[pl.ds(0, 256)])

        def elu_loop(r, c):
            for half in range(2):
                v = rows[r, pl.ds(half * 16, 16)]
                ev = jnp.exp(v) - 1.0
                rows[r, pl.ds(half * 16, 16)] = jnp.where(v > 0.0, v, ev)
            return c
        lax.fori_loop(0, 256, elu_loop, 0)
        pltpu.sync_copy(rows.at[pl.ds(0, 256)], outc.at[cid, pl.ds(rb, 256)])


_sc_kernel = functools.partial(
    pl.kernel,
    out_type=jax.ShapeDtypeStruct((2, 20480, C), F32),
    mesh=plsc.VectorSubcoreMesh(core_axis_name="c", subcore_axis_name="s"),
    compiler_params=pltpu.CompilerParams(needs_layout_passes=False, use_tc_tiling_on_sc=False),
    scratch_types=[
        pltpu.VMEM_SHARED((20480, C), F32),
        pltpu.VMEM_SHARED((1280, 16), F32),
        pltpu.VMEM((4 * NP1,), F32),
        pltpu.VMEM((1280, 16), F32),
        pltpu.VMEM((BLK, C), F32),
        pltpu.VMEM((SBLK,), I32),
        pltpu.VMEM((SBLK,), I32),
        pltpu.VMEM((BLK,), I32),
        pltpu.VMEM((BLK,), I32),
        pltpu.VMEM((BLK,), F32),
        pltpu.VMEM((32,), F32),
        pltpu.VMEM((640,), I32),
    ],
)(_sc_body)


def kernel(x, edge_index, W, att):
    att_s = att[0, :, :C]
    att_d = att[0, :, C:]
    W3 = W.reshape(D, H, C)
    Ws = (W3 * att_s[None, :, :]).sum(-1)
    Wd = (W3 * att_d[None, :, :]).sum(-1)
    wcat = jnp.concatenate(
        [W, Ws, Wd, jnp.zeros((D, 256 - D - 2 * H), F32)], axis=1)
    xpad = jnp.concatenate([x, jnp.zeros((NPAD - N, D), F32)], axis=0)

    y = _project(xpad, wcat)
    wh = y[:N, :D]
    s = y[:N, D:D + H]
    d = y[:N, D + H:D + 2 * H]

    sdt = jnp.concatenate([s.T, d.T], axis=0).reshape(8 * N)
    mh = s.max(0) + d.max(0)
    mh = jnp.maximum(mh, 0.2 * mh)
    mtab = jnp.repeat(mh, 16)
    whr = wh.reshape(N * H, C)
    iden = jnp.arange(1280, dtype=I32)

    outc = _sc_kernel(edge_index[0], edge_index[1], sdt, whr, mtab, iden)
    return jnp.concatenate(
        [outc[0, :2 * N].reshape(N, 2 * C),
         outc[1, :2 * N].reshape(N, 2 * C)], axis=1)
